# Initial kernel scaffold; baseline (speedup 1.0000x reference)
import functools

import jax
import jax.numpy as jnp
from jax import lax
from jax.experimental import pallas as pl
from jax.experimental.pallas import tpu as pltpu
from jax.experimental.pallas import tpu_sc as plsc

DIM = 256
LANES = 16
CHUNK = 128


@functools.lru_cache(maxsize=None)
def _make_sc_kernel(vocab, batch):
    info = plsc.get_sparse_core_info()
    nc, ns = info.num_cores, info.num_subcores
    nw = nc * ns
    assert batch % (nw * CHUNK) == 0
    b_per_w = batch // nw
    nchunks = b_per_w // CHUNK
    nvec = DIM // LANES
    mesh = plsc.VectorSubcoreMesh(core_axis_name="c", subcore_axis_name="s")

    @functools.partial(
        pl.kernel,
        mesh=mesh,
        out_type=jax.ShapeDtypeStruct((batch,), jnp.float32),
        scratch_types=[
            pltpu.VMEM((CHUNK,), jnp.int32),
            pltpu.VMEM((CHUNK,), jnp.int32),
            pltpu.VMEM((CHUNK, DIM), jnp.float32),
            pltpu.VMEM((CHUNK, DIM), jnp.float32),
            pltpu.VMEM((DIM,), jnp.float32),
            pltpu.VMEM((LANES,), jnp.float32),
            pltpu.VMEM((LANES * LANES,), jnp.float32),
            pltpu.VMEM((b_per_w,), jnp.float32),
            pltpu.SemaphoreType.DMA,
            pltpu.SemaphoreType.DMA,
        ],
    )
    def k(table_hbm, idx_hbm, w_hbm, b_hbm, out_hbm,
          idx0, idx1, rows0, rows1, w_v, b_v, ps, out_v, sem0, sem1):
        wid = lax.axis_index("s") * nc + lax.axis_index("c")
        base = wid * b_per_w

        pltpu.sync_copy(w_hbm, w_v)
        pltpu.sync_copy(b_hbm, b_v)

        idx_bufs = (idx0, idx1)
        row_bufs = (rows0, rows1)
        sems = (sem0, sem1)

        w_regs = [w_v[pl.ds(LANES * j, LANES)] for j in range(nvec)]
        b_reg = b_v[...]
        iota16 = lax.iota(jnp.int32, LANES) * LANES

        def start(c):
            s = c % 2
            pltpu.sync_copy(idx_hbm.at[pl.ds(base + c * CHUNK, CHUNK)],
                            idx_bufs[s])
            return pltpu.async_copy(table_hbm.at[idx_bufs[s]], row_bufs[s],
                                    sems[s])

        copies = [None, None]
        copies[0] = start(0)

        for c in range(nchunks):
            s = c % 2
            if c + 1 < nchunks:
                copies[(c + 1) % 2] = start(c + 1)
            copies[s].wait()
            rows = row_bufs[s]

            def group(g, _, rows=rows, c=c):
                for r in range(LANES):
                    row = g * LANES + r
                    p = rows[row, pl.ds(0, LANES)] * w_regs[0]
                    for j in range(1, nvec):
                        p = p + rows[row, pl.ds(LANES * j, LANES)] * w_regs[j]
                    ps[pl.ds(r * LANES, LANES)] = p
                acc = plsc.load_gather(ps, [iota16])
                for l in range(1, LANES):
                    acc = acc + plsc.load_gather(ps, [iota16 + l])
                logits = acc + b_reg
                y = 1.0 / (1.0 + jnp.exp(-logits))
                out_v[pl.ds(c * CHUNK + g * LANES, LANES)] = y
                return 0

            lax.fori_loop(0, CHUNK // LANES, group, 0)

        pltpu.sync_copy(out_v, out_hbm.at[pl.ds(base, b_per_w)])

    return k


def kernel(inputs, embedding_0, W, b):
    batch = inputs.shape[0]
    vocab = embedding_0.shape[0]
    idx = inputs.reshape(batch).astype(jnp.int32)
    w_flat = W.reshape(DIM).astype(jnp.float32)
    b_vec = jnp.broadcast_to(b.reshape(1).astype(jnp.float32), (LANES,))
    out = _make_sc_kernel(vocab, batch)(embedding_0, idx, w_flat, b_vec)
    return out.reshape(batch, 1)


# speedup vs baseline: 6.5994x; 6.5994x over previous
import functools

import jax
import jax.numpy as jnp
from jax import lax
from jax.experimental import pallas as pl
from jax.experimental.pallas import tpu as pltpu
from jax.experimental.pallas import tpu_sc as plsc

DIM = 256
LANES = 16
CHUNK = 128


@functools.lru_cache(maxsize=None)
def _make_sc_kernel(vocab, batch):
    info = plsc.get_sparse_core_info()
    nc, ns = info.num_cores, info.num_subcores
    nw = nc * ns
    assert batch % (nw * CHUNK) == 0
    b_per_w = batch // nw
    nchunks = b_per_w // CHUNK
    nvec = DIM // LANES
    mesh = plsc.VectorSubcoreMesh(core_axis_name="c", subcore_axis_name="s")

    @functools.partial(
        pl.kernel,
        mesh=mesh,
        out_type=jax.ShapeDtypeStruct((batch,), jnp.float32),
        compiler_params=pltpu.CompilerParams(needs_layout_passes=False),
        scratch_types=[
            pltpu.VMEM((CHUNK,), jnp.int32),
            pltpu.VMEM((CHUNK,), jnp.int32),
            pltpu.VMEM((CHUNK, DIM), jnp.float32),
            pltpu.VMEM((CHUNK, DIM), jnp.float32),
            pltpu.VMEM((DIM,), jnp.float32),
            pltpu.VMEM((LANES,), jnp.float32),
            pltpu.VMEM((LANES * LANES,), jnp.float32),
            pltpu.VMEM((b_per_w,), jnp.float32),
            pltpu.SemaphoreType.DMA,
            pltpu.SemaphoreType.DMA,
        ],
    )
    def k(table_hbm, idx_hbm, w_hbm, b_hbm, out_hbm,
          idx0, idx1, rows0, rows1, w_v, b_v, ps, out_v, sem0, sem1):
        wid = lax.axis_index("s") * nc + lax.axis_index("c")
        base = wid * b_per_w

        pltpu.sync_copy(w_hbm, w_v)
        pltpu.sync_copy(b_hbm, b_v)

        idx_bufs = (idx0, idx1)
        row_bufs = (rows0, rows1)
        sems = (sem0, sem1)

        w_regs = [w_v[pl.ds(LANES * j, LANES)] for j in range(nvec)]
        b_reg = b_v[...]
        iota16 = lax.iota(jnp.int32, LANES) * LANES

        def start(c):
            s = c % 2
            pltpu.sync_copy(idx_hbm.at[pl.ds(base + c * CHUNK, CHUNK)],
                            idx_bufs[s])
            return pltpu.async_copy(table_hbm.at[idx_bufs[s]], row_bufs[s],
                                    sems[s])

        copies = [None, None]
        copies[0] = start(0)

        for c in range(nchunks):
            s = c % 2
            if c + 1 < nchunks:
                copies[(c + 1) % 2] = start(c + 1)
            copies[s].wait()
            rows = row_bufs[s]

            def group(g, _, rows=rows, c=c):
                for r in range(LANES):
                    row = g * LANES + r
                    p = rows[row, pl.ds(0, LANES)] * w_regs[0]
                    for j in range(1, nvec):
                        p = p + rows[row, pl.ds(LANES * j, LANES)] * w_regs[j]
                    ps[pl.ds(r * LANES, LANES)] = p
                acc = plsc.load_gather(ps, [iota16])
                for l in range(1, LANES):
                    acc = acc + plsc.load_gather(ps, [iota16 + l])
                logits = acc + b_reg
                y = 1.0 / (1.0 + jnp.exp(-logits))
                out_v[pl.ds(c * CHUNK + g * LANES, LANES)] = y
                return 0

            lax.fori_loop(0, CHUNK // LANES, group, 0)

        pltpu.sync_copy(out_v, out_hbm.at[pl.ds(base, b_per_w)])

    return k


def kernel(inputs, embedding_0, W, b):
    batch = inputs.shape[0]
    vocab = embedding_0.shape[0]
    idx = inputs.reshape(batch).astype(jnp.int32)
    w_flat = W.reshape(DIM).astype(jnp.float32)
    b_vec = jnp.broadcast_to(b.reshape(1).astype(jnp.float32), (LANES,))
    out = _make_sc_kernel(vocab, batch)(embedding_0, idx, w_flat, b_vec)
    return out.reshape(batch, 1)


# speedup vs baseline: 7.0586x; 1.0696x over previous
import functools

import jax
import jax.numpy as jnp
from jax import lax
from jax.experimental import pallas as pl
from jax.experimental.pallas import tpu as pltpu
from jax.experimental.pallas import tpu_sc as plsc

DIM = 256
LANES = 16
CHUNK = 128
NBUF = 3


def _perm(v, idx):
    return jnp.take_along_axis(v, idx, axis=0, mode="promise_in_bounds")


@functools.lru_cache(maxsize=None)
def _make_sc_kernel(vocab, batch):
    info = plsc.get_sparse_core_info()
    nc, ns = info.num_cores, info.num_subcores
    nw = nc * ns
    assert batch % (nw * CHUNK) == 0
    b_per_w = batch // nw
    nchunks = b_per_w // CHUNK
    nvec = DIM // LANES
    mesh = plsc.VectorSubcoreMesh(core_axis_name="c", subcore_axis_name="s")

    @functools.partial(
        pl.kernel,
        mesh=mesh,
        out_type=jax.ShapeDtypeStruct((batch,), jnp.float32),
        compiler_params=pltpu.CompilerParams(needs_layout_passes=False),
        scratch_types=[
            pltpu.VMEM((b_per_w,), jnp.int32),
            pltpu.VMEM((CHUNK, DIM), jnp.float32),
            pltpu.VMEM((CHUNK, DIM), jnp.float32),
            pltpu.VMEM((CHUNK, DIM), jnp.float32),
            pltpu.VMEM((DIM,), jnp.float32),
            pltpu.VMEM((LANES,), jnp.float32),
            pltpu.VMEM((b_per_w,), jnp.float32),
            pltpu.SemaphoreType.DMA,
            pltpu.SemaphoreType.DMA,
            pltpu.SemaphoreType.DMA,
        ],
    )
    def k(table_hbm, idx_hbm, w_hbm, b_hbm, out_hbm,
          idx_v, rows0, rows1, rows2, w_v, b_v, out_v, sem0, sem1, sem2):
        wid = lax.axis_index("s") * nc + lax.axis_index("c")
        base = wid * b_per_w

        pltpu.sync_copy(idx_hbm.at[pl.ds(base, b_per_w)], idx_v)
        pltpu.sync_copy(w_hbm, w_v)
        pltpu.sync_copy(b_hbm, b_v)

        row_bufs = (rows0, rows1, rows2)
        sems = (sem0, sem1, sem2)

        w_regs = [w_v[pl.ds(LANES * j, LANES)] for j in range(nvec)]
        b_reg = b_v[...]
        iota = lax.iota(jnp.int32, LANES)
        xors = {h: iota ^ h for h in (8, 4, 2, 1)}
        masks = {h: (iota & h) == 0 for h in (8, 4, 2, 1)}
        bitrev = (((iota & 1) << 3) | ((iota & 2) << 1)
                  | ((iota & 4) >> 1) | ((iota & 8) >> 3))

        def start(c):
            s = c % NBUF
            return pltpu.async_copy(
                table_hbm.at[idx_v.at[pl.ds(c * CHUNK, CHUNK)]],
                row_bufs[s], sems[s])

        copies = [None] * NBUF
        for c in range(min(NBUF, nchunks)):
            copies[c % NBUF] = start(c)

        for c in range(nchunks):
            s = c % NBUF
            copies[s].wait()
            rows = row_bufs[s]

            def group(g, _, rows=rows, c=c):
                vecs = []
                for r in range(LANES):
                    row = g * LANES + r
                    a0 = rows[row, pl.ds(0, LANES)] * w_regs[0]
                    a1 = rows[row, pl.ds(LANES, LANES)] * w_regs[1]
                    for j in range(2, nvec, 2):
                        a0 = a0 + rows[row, pl.ds(LANES * j, LANES)] * w_regs[j]
                        a1 = a1 + rows[row, pl.ds(LANES * (j + 1), LANES)] * w_regs[j + 1]
                    vecs.append(a0 + a1)
                for h in (8, 4, 2, 1):
                    nxt = []
                    for i in range(0, len(vecs), 2):
                        x, y = vecs[i], vecs[i + 1]
                        t1 = jnp.where(masks[h], x, _perm(y, xors[h]))
                        t2 = jnp.where(masks[h], _perm(x, xors[h]), y)
                        nxt.append(t1 + t2)
                    vecs = nxt
                logits = _perm(vecs[0], bitrev) + b_reg
                y = 1.0 / (1.0 + jnp.exp(-logits))
                out_v[pl.ds(c * CHUNK + g * LANES, LANES)] = y
                return 0

            lax.fori_loop(0, CHUNK // LANES, group, 0)
            if c + NBUF < nchunks:
                copies[s] = start(c + NBUF)

        pltpu.sync_copy(out_v, out_hbm.at[pl.ds(base, b_per_w)])

    return k


def kernel(inputs, embedding_0, W, b):
    batch = inputs.shape[0]
    vocab = embedding_0.shape[0]
    idx = inputs.reshape(batch).astype(jnp.int32)
    w_flat = W.reshape(DIM).astype(jnp.float32)
    b_vec = jnp.broadcast_to(b.reshape(1).astype(jnp.float32), (LANES,))
    out = _make_sc_kernel(vocab, batch)(embedding_0, idx, w_flat, b_vec)
    return out.reshape(batch, 1)
